# baseline (speedup 1.0000x reference)
import functools

import jax
import jax.numpy as jnp
from jax import lax
from jax.experimental import pallas as pl
from jax.experimental.pallas import tpu as pltpu
from jax.experimental.pallas import tpu_sc as plsc

DIM = 32
LANES = 16

_info = plsc.get_sparse_core_info()
NC = _info.num_cores
NS = _info.num_subcores
NW = NC * NS

ICHUNK = 128


def _sc_body(uid_hbm, iid_hbm, umat, imat, out_hbm,
             uid_v, iid_v, u_rows, i_rows, scores_v, sem, b_per_w):
    wid = lax.axis_index("s") * NC + lax.axis_index("c")
    base = wid * b_per_w

    pltpu.sync_copy(uid_hbm.at[pl.ds(base, b_per_w)], uid_v)
    pltpu.sync_copy(iid_hbm.at[pl.ds(base, b_per_w)], iid_v)

    nch = b_per_w // ICHUNK
    for c in range(nch):
        sl = pl.ds(c * ICHUNK, ICHUNK)
        pltpu.async_copy(umat.at[uid_v.at[sl]], u_rows.at[sl, :], sem)
        pltpu.async_copy(imat.at[iid_v.at[sl]], i_rows.at[sl, :], sem)
    for c in range(nch):
        sl = pl.ds(c * ICHUNK, ICHUNK)
        pltpu.make_async_copy(umat.at[uid_v.at[sl]], u_rows.at[sl, :], sem).wait()
        pltpu.make_async_copy(imat.at[iid_v.at[sl]], i_rows.at[sl, :], sem).wait()

    iota = lax.iota(jnp.int32, LANES)

    def blk_body(blk, _):
        row_idx = blk * LANES + iota
        acc = jnp.zeros((LANES,), jnp.float32)
        for d in range(DIM):
            col = jnp.full((LANES,), d, jnp.int32)
            u = plsc.load_gather(u_rows, [row_idx, col])
            v = plsc.load_gather(i_rows, [row_idx, col])
            acc = acc + u * v
        scores_v[pl.ds(blk * LANES, LANES)] = acc
        return 0

    lax.fori_loop(0, b_per_w // LANES, blk_body, 0)

    pltpu.sync_copy(scores_v, out_hbm.at[pl.ds(base, b_per_w)])


def kernel(uid, iid, user_matrix, item_matrix):
    B = uid.shape[0]
    b_per_w = B // NW

    mesh = plsc.VectorSubcoreMesh(core_axis_name="c", subcore_axis_name="s")

    sc_call = functools.partial(
        pl.kernel,
        mesh=mesh,
        compiler_params=pltpu.CompilerParams(
            needs_layout_passes=False, use_tc_tiling_on_sc=False),
        out_type=jax.ShapeDtypeStruct((B,), jnp.float32),
        scratch_types=[
            pltpu.VMEM((b_per_w,), jnp.int32),
            pltpu.VMEM((b_per_w,), jnp.int32),
            pltpu.VMEM((b_per_w, DIM), jnp.float32),
            pltpu.VMEM((b_per_w, DIM), jnp.float32),
            pltpu.VMEM((b_per_w,), jnp.float32),
            pltpu.SemaphoreType.DMA,
        ],
    )(functools.partial(_sc_body, b_per_w=b_per_w))

    return sc_call(uid, iid, user_matrix, item_matrix)


# speedup vs baseline: 1.0029x; 1.0029x over previous
import functools

import jax
import jax.numpy as jnp
from jax import lax
from jax.experimental import pallas as pl
from jax.experimental.pallas import tpu as pltpu
from jax.experimental.pallas import tpu_sc as plsc

DIM = 32
LANES = 16
ICHUNK = 128

_info = plsc.get_sparse_core_info()
NC = _info.num_cores
NS = _info.num_subcores
NW = NC * NS


def _sc_body(uid_hbm, iid_hbm, umat, imat, out_hbm,
             uid_v, iid_v, urows, irows, scores_v, sem,
             b_per_w):
    wid = lax.axis_index("s") * NC + lax.axis_index("c")
    base = wid * b_per_w

    pltpu.sync_copy(uid_hbm.at[pl.ds(base, b_per_w)], uid_v)
    pltpu.sync_copy(iid_hbm.at[pl.ds(base, b_per_w)], iid_v)

    nch = b_per_w // ICHUNK
    for c in range(nch):
        sl = pl.ds(c * ICHUNK, ICHUNK)
        pltpu.async_copy(umat.at[uid_v.at[sl]], urows.at[sl, :], sem)
        pltpu.async_copy(imat.at[iid_v.at[sl]], irows.at[sl, :], sem)
    for c in range(nch):
        sl = pl.ds(c * ICHUNK, ICHUNK)
        pltpu.make_async_copy(umat.at[uid_v.at[sl]], urows.at[sl, :], sem).wait()
        pltpu.make_async_copy(imat.at[iid_v.at[sl]], irows.at[sl, :], sem).wait()

    def dot(b, _):
        acc = jnp.zeros((LANES,), jnp.float32)
        rows = b * LANES + lax.iota(jnp.int32, LANES)
        for d in range(DIM):
            cold = jnp.full((LANES,), d, jnp.int32)
            u = plsc.load_gather(urows, [rows, cold])
            v = plsc.load_gather(irows, [rows, cold])
            acc = acc + u * v
        scores_v[pl.ds(b * LANES, LANES)] = acc
        return 0

    lax.fori_loop(0, b_per_w // LANES, dot, 0)

    pltpu.sync_copy(scores_v, out_hbm.at[pl.ds(base, b_per_w)])


def kernel(uid, iid, user_matrix, item_matrix):
    B = uid.shape[0]
    b_per_w = B // NW

    mesh = plsc.VectorSubcoreMesh(core_axis_name="c", subcore_axis_name="s")

    sc_call = functools.partial(
        pl.kernel,
        mesh=mesh,
        compiler_params=pltpu.CompilerParams(
            needs_layout_passes=False, use_tc_tiling_on_sc=False),
        out_type=jax.ShapeDtypeStruct((B,), jnp.float32),
        scratch_types=[
            pltpu.VMEM((b_per_w,), jnp.int32),
            pltpu.VMEM((b_per_w,), jnp.int32),
            pltpu.VMEM((b_per_w, DIM), jnp.float32),
            pltpu.VMEM((b_per_w, DIM), jnp.float32),
            pltpu.VMEM((b_per_w,), jnp.float32),
            pltpu.SemaphoreType.DMA,
        ],
    )(functools.partial(_sc_body, b_per_w=b_per_w))

    return sc_call(uid, iid, user_matrix, item_matrix)
